# Initial kernel scaffold; baseline (speedup 1.0000x reference)
import jax
import jax.numpy as jnp
from jax.experimental import pallas as pl


def kernel(x, edge_index, W1, b1, W2, b2):
    raise NotImplementedError("write your pallas kernel here")


# speedup vs baseline: 6.4755x; 6.4755x over previous
import functools

import jax
import jax.numpy as jnp
from jax import lax
from jax.experimental import pallas as pl
from jax.experimental.pallas import tpu as pltpu
from jax.experimental.pallas import tpu_sc as plsc

N_NODES = 10000
N_EDGES = 320000
D = 128

NC = 2
NS = 16
NW = NC * NS

CHUNK = 128
NCHUNK = N_EDGES // CHUNK
MAX_ITER = -(-NCHUNK // NW)
ROWS_PER_SUB = 624
TAIL_ROWS = N_NODES - NS * ROWS_PER_SUB


def _sc_aggregate(x, src, dst, zeros):
    mesh = plsc.VectorSubcoreMesh(core_axis_name="c", subcore_axis_name="s")

    @functools.partial(
        pl.kernel,
        mesh=mesh,
        out_type=jax.ShapeDtypeStruct((NC, N_NODES, D), jnp.float32),
        scratch_types=[
            pltpu.VMEM((CHUNK,), jnp.int32),
            pltpu.VMEM((CHUNK,), jnp.int32),
            pltpu.VMEM((CHUNK, D), jnp.float32),
            pltpu.VMEM_SHARED((N_NODES, D), jnp.float32),
            pltpu.SemaphoreType.DMA,
        ],
    )
    def agg(x_hbm, src_hbm, dst_hbm, zeros_hbm, out_hbm,
            src_v, dst_v, rows_v, acc, sem):
        c = lax.axis_index("c")
        s = lax.axis_index("s")
        wid = s * NC + c

        row0 = s * ROWS_PER_SUB
        pltpu.sync_copy(zeros_hbm.at[pl.ds(row0, ROWS_PER_SUB)],
                        acc.at[pl.ds(row0, ROWS_PER_SUB)])

        @pl.when(s == NS - 1)
        def _():
            t0 = NS * ROWS_PER_SUB
            pltpu.sync_copy(zeros_hbm.at[pl.ds(t0, TAIL_ROWS)],
                            acc.at[pl.ds(t0, TAIL_ROWS)])

        plsc.subcore_barrier()

        def body(i, carry):
            chunk = wid + NW * i

            @pl.when(chunk < NCHUNK)
            def _():
                off = chunk * CHUNK
                pltpu.sync_copy(src_hbm.at[pl.ds(off, CHUNK)], src_v)
                pltpu.sync_copy(dst_hbm.at[pl.ds(off, CHUNK)], dst_v)
                pltpu.async_copy(x_hbm.at[src_v], rows_v, sem).wait()
                pltpu.sync_copy(rows_v, acc.at[dst_v], add=True)

            return carry

        lax.fori_loop(0, MAX_ITER, body, 0)
        plsc.subcore_barrier()

        pltpu.sync_copy(acc.at[pl.ds(row0, ROWS_PER_SUB)],
                        out_hbm.at[c, pl.ds(row0, ROWS_PER_SUB)])

        @pl.when(s == NS - 1)
        def _():
            t0 = NS * ROWS_PER_SUB
            pltpu.sync_copy(acc.at[pl.ds(t0, TAIL_ROWS)],
                            out_hbm.at[c, pl.ds(t0, TAIL_ROWS)])

    return agg(x, src, dst, zeros)


def _mlp_block(x_ref, p0_ref, p1_ref, w1_ref, b1_ref, w2_ref, b2_ref, o_ref):
    h = x_ref[...] + p0_ref[...] + p1_ref[...]
    h = jnp.dot(h, w1_ref[...], preferred_element_type=jnp.float32) + b1_ref[...]
    h = jnp.maximum(h, 0.0)
    o_ref[...] = (
        jnp.dot(h, w2_ref[...], preferred_element_type=jnp.float32) + b2_ref[...]
    )


def _tc_mlp(x, p0, p1, W1, b1, W2, b2):
    blk = 1000
    grid = (N_NODES // blk,)
    row_spec = pl.BlockSpec((blk, D), lambda i: (i, 0))
    full_spec = pl.BlockSpec((D, D), lambda i: (0, 0))
    bias_spec = pl.BlockSpec((1, D), lambda i: (0, 0))
    return pl.pallas_call(
        _mlp_block,
        grid=grid,
        in_specs=[row_spec, row_spec, row_spec,
                  full_spec, bias_spec, full_spec, bias_spec],
        out_specs=row_spec,
        out_shape=jax.ShapeDtypeStruct((N_NODES, D), jnp.float32),
    )(x, p0, p1, W1.T, b1.reshape(1, D), W2.T, b2.reshape(1, D))


def kernel(x, edge_index, W1, b1, W2, b2):
    src = edge_index[0].astype(jnp.int32)
    dst = edge_index[1].astype(jnp.int32)
    zeros = jnp.zeros((N_NODES, D), jnp.float32)
    partials = _sc_aggregate(x, src, dst, zeros)
    return _tc_mlp(x, partials[0], partials[1], W1, b1, W2, b2)
